# baseline (speedup 1.0000x reference)
import functools

import jax
import jax.numpy as jnp
from jax import lax
from jax.experimental import pallas as pl
from jax.experimental.pallas import tpu as pltpu
from jax.experimental.pallas import tpu_sc as plsc

_N_FIELDS = 26
_VOCAB = 100000
_D = 50
_NC = 2
_NS = 16
_NW = _NC * _NS
_L = 16
_RPD = 128
_CHUNK = 1024


@functools.cache
def _gather_call(b_total):
    b_per_w = b_total // _NW
    n_chunks = b_per_w // _CHUNK
    n_dma = _CHUNK // _RPD
    mesh = plsc.VectorSubcoreMesh(core_axis_name="c", subcore_axis_name="s")

    @functools.partial(
        pl.kernel,
        out_type=jax.ShapeDtypeStruct((b_total, _D), jnp.float32),
        mesh=mesh,
        scratch_types=[
            pltpu.VMEM((_CHUNK,), jnp.int32),
            pltpu.VMEM((n_dma, _RPD), jnp.int32),
            pltpu.VMEM((_CHUNK, _D), jnp.float32),
            pltpu.SemaphoreType.DMA,
        ],
        compiler_params=pltpu.CompilerParams(use_tc_tiling_on_sc=False),
    )
    def k(tab_hbm, idx_hbm, out_hbm, idx_raw, idx2d, rows, sem):
        wid = lax.axis_index("s") * _NC + lax.axis_index("c")
        base = wid * b_per_w
        iota = lax.iota(jnp.int32, _L)

        def chunk_body(c, carry):
            start = base + c * _CHUNK
            pltpu.sync_copy(idx_hbm.at[pl.ds(start, _CHUNK)], idx_raw)
            for i in range(_CHUNK // _L):
                pos = start + i * _L + iota
                f = lax.rem(pos, _N_FIELDS)
                g = idx_raw[pl.ds(i * _L, _L)] + f * _VOCAB
                idx2d[i // (_RPD // _L), pl.ds((i % (_RPD // _L)) * _L, _L)] = g
            descs = [
                pltpu.async_copy(
                    tab_hbm.at[idx2d.at[j]],
                    rows.at[pl.ds(j * _RPD, _RPD)],
                    sem,
                )
                for j in range(n_dma)
            ]
            for d in descs:
                d.wait()
            pltpu.sync_copy(rows, out_hbm.at[pl.ds(start, _CHUNK)])
            return carry

        lax.fori_loop(0, n_chunks, chunk_body, 0)

    return k


def kernel(cat, tables):
    batch, n_fields = cat.shape
    _, vocab, d = tables.shape
    tab = tables.reshape(n_fields * vocab, d)
    idx = cat.reshape(-1)
    out = _gather_call(batch * n_fields)(tab, idx)
    return out.reshape(batch, n_fields * d)


# speedup vs baseline: 1.0008x; 1.0008x over previous
import functools

import jax
import jax.numpy as jnp
from jax import lax
from jax.experimental import pallas as pl
from jax.experimental.pallas import tpu as pltpu
from jax.experimental.pallas import tpu_sc as plsc

_N_FIELDS = 26
_VOCAB = 100000
_DPAD = 56
_NC = 2
_NS = 16
_NW = _NC * _NS
_L = 16
_RPD = 128
_CHUNK = 1024


@functools.cache
def _gather_call(b_total):
    b_per_w = b_total // _NW
    n_chunks = b_per_w // _CHUNK
    n_dma = _CHUNK // _RPD
    mesh = plsc.VectorSubcoreMesh(core_axis_name="c", subcore_axis_name="s")

    @functools.partial(
        pl.kernel,
        out_type=jax.ShapeDtypeStruct((b_total, _DPAD), jnp.float32),
        mesh=mesh,
        scratch_types=[
            pltpu.VMEM((_CHUNK,), jnp.int32),
            pltpu.VMEM((n_dma, _RPD), jnp.int32),
            pltpu.VMEM((_CHUNK, _DPAD), jnp.float32),
            pltpu.SemaphoreType.DMA,
        ],
        compiler_params=pltpu.CompilerParams(use_tc_tiling_on_sc=False),
    )
    def k(tab_hbm, idx_hbm, out_hbm, idx_raw, idx2d, rows, sem):
        wid = lax.axis_index("s") * _NC + lax.axis_index("c")
        base = wid * b_per_w
        iota = lax.iota(jnp.int32, _L)

        def chunk_body(c, carry):
            start = base + c * _CHUNK
            pltpu.sync_copy(idx_hbm.at[pl.ds(start, _CHUNK)], idx_raw)
            for i in range(_CHUNK // _L):
                pos = start + i * _L + iota
                f = lax.rem(pos, _N_FIELDS)
                g = idx_raw[pl.ds(i * _L, _L)] + f * _VOCAB
                idx2d[i // (_RPD // _L), pl.ds((i % (_RPD // _L)) * _L, _L)] = g
            descs = [
                pltpu.async_copy(
                    tab_hbm.at[idx2d.at[j]],
                    rows.at[pl.ds(j * _RPD, _RPD)],
                    sem,
                )
                for j in range(n_dma)
            ]
            for d in descs:
                d.wait()
            pltpu.sync_copy(rows, out_hbm.at[pl.ds(start, _CHUNK)])
            return carry

        lax.fori_loop(0, n_chunks, chunk_body, 0)

    return k


def kernel(cat, tables):
    batch, n_fields = cat.shape
    _, vocab, d = tables.shape
    tab = jnp.pad(tables, ((0, 0), (0, 0), (0, _DPAD - d)))
    tab = tab.reshape(n_fields * vocab, _DPAD)
    idx = cat.reshape(-1)
    out = _gather_call(batch * n_fields)(tab, idx)
    return out[:, :d].reshape(batch, n_fields * d)
